# baseline (speedup 1.0000x reference)
import functools

import jax
import jax.numpy as jnp
from jax import lax
from jax.experimental import pallas as pl
from jax.experimental.pallas import tpu as pltpu
from jax.experimental.pallas import tpu_sc as plsc

B, N, C, K, NUM_CLASSES = 2, 4096, 128, 16, 1000
CLS_PAD = 1024
ROWS_B = 256
ROWS_D = 512
BIG = 3.0e38


def _cpe_fc1_body(x_ref, cpe_w_ref, cpe_b_ref, fc1_w_ref, fc1_b_ref,
                  xc_ref, h_ref):
    x = x_ref[0]
    w0 = cpe_w_ref[0:1, :]
    w1 = cpe_w_ref[1:2, :]
    w2 = cpe_w_ref[2:3, :]
    zrow = jnp.zeros((1, C), jnp.float32)
    xm1 = jnp.concatenate([zrow, x[:-1, :]], axis=0)
    xp1 = jnp.concatenate([x[1:, :], zrow], axis=0)
    conv = xm1 * w0 + x * w1 + xp1 * w2 + cpe_b_ref[...]
    xc = x + conv
    xc_ref[0] = xc
    h_ref[0] = jnp.dot(xc, fc1_w_ref[...],
                       preferred_element_type=jnp.float32) + fc1_b_ref[...]


def _cpe_fc1(x, cpe_w, cpe_b2, fc1_w, fc1_b2):
    return pl.pallas_call(
        _cpe_fc1_body,
        grid=(B,),
        in_specs=[
            pl.BlockSpec((1, N, C), lambda b: (b, 0, 0)),
            pl.BlockSpec((3, C), lambda b: (0, 0)),
            pl.BlockSpec((1, C), lambda b: (0, 0)),
            pl.BlockSpec((C, C), lambda b: (0, 0)),
            pl.BlockSpec((1, C), lambda b: (0, 0)),
        ],
        out_specs=[
            pl.BlockSpec((1, N, C), lambda b: (b, 0, 0)),
            pl.BlockSpec((1, N, C), lambda b: (b, 0, 0)),
        ],
        out_shape=[
            jax.ShapeDtypeStruct((B, N, C), jnp.float32),
            jax.ShapeDtypeStruct((B, N, C), jnp.float32),
        ],
    )(x, cpe_w, cpe_b2, fc1_w, fc1_b2)


def _topk_body(hrow_ref, hfull_ref, idx_ref, d_ref):
    hb = hrow_ref[...]
    h = hfull_ref[...]
    sqcol = jnp.sum(h * h, axis=1, keepdims=True)
    lhs = jnp.concatenate([hb * (-2.0), jnp.ones((ROWS_B, 1), jnp.float32)],
                          axis=1)
    rhs = jnp.concatenate([h, sqcol], axis=1)
    e = lax.dot_general(lhs, rhs, (((1,), (1,)), ((), ())),
                        preferred_element_type=jnp.float32)
    d_ref[...] = e
    iota = lax.broadcasted_iota(jnp.int32, (ROWS_B, N), 1).astype(jnp.float32)
    cols = []
    for _ in range(K // 2):
        d = d_ref[...]
        m1 = jnp.min(d, axis=1, keepdims=True)
        c1 = jnp.min(jnp.where(d <= m1, iota, jnp.float32(1e9)),
                     axis=1, keepdims=True)
        d = jnp.where(iota == c1, BIG, d)
        m2 = jnp.min(d, axis=1, keepdims=True)
        c2 = jnp.min(jnp.where(d <= m2, iota, jnp.float32(1e9)),
                     axis=1, keepdims=True)
        d_ref[...] = jnp.where(iota == c2, BIG, d)
        cols += [c1, c2]
    pad = jnp.zeros((ROWS_B, 128 - K), jnp.float32)
    idx_mat = jnp.concatenate(cols + [pad], axis=1).astype(jnp.int32)
    idx_ref[...] = idx_mat


def _knn_topk(h):
    return pl.pallas_call(
        _topk_body,
        grid=(N // ROWS_B,),
        in_specs=[
            pl.BlockSpec((ROWS_B, C), lambda i: (i, 0)),
            pl.BlockSpec((N, C), lambda i: (0, 0)),
        ],
        out_specs=pl.BlockSpec((ROWS_B, 128), lambda i: (i, 0)),
        out_shape=jax.ShapeDtypeStruct((N, 128), jnp.int32),
        scratch_shapes=[pltpu.VMEM((ROWS_B, N), jnp.float32)],
    )(h, h)


CH = 16
LANES = 16


@functools.lru_cache(maxsize=None)
def _make_gather_rel():
    info = plsc.get_sparse_core_info()
    num_cores = info.num_cores
    nw = num_cores * info.num_subcores
    rows_w = N // nw

    nch = rows_w // CH

    @functools.partial(
        pl.kernel,
        out_type=jax.ShapeDtypeStruct((N, C), jnp.float32),
        mesh=plsc.VectorSubcoreMesh(core_axis_name="c",
                                    subcore_axis_name="s"),
        scratch_types=[
            pltpu.VMEM((2, CH, 128), jnp.int32),
            pltpu.VMEM((2, CH * K), jnp.int32),
            pltpu.VMEM((2, CH * K, C), jnp.float32),
            pltpu.VMEM((2, CH, C), jnp.float32),
            pltpu.VMEM((CH, C), jnp.float32),
            pltpu.SemaphoreType.DMA,
            pltpu.SemaphoreType.DMA,
            pltpu.SemaphoreType.DMA,
            pltpu.SemaphoreType.DMA,
        ],
    )
    def gather_rel(h_hbm, idx_hbm, rel_hbm, idxp_v, idx_v, rows_v, hrow_v,
                   out_v, gsem0, gsem1, hsem0, hsem1):
        gsem = (gsem0, gsem1)
        hsem = (hsem0, hsem1)
        wid = lax.axis_index("s") * num_cores + lax.axis_index("c")
        w0 = wid * rows_w

        half = CH * K // 2

        def fetch(t, buf):
            base = w0 + t * CH
            pltpu.sync_copy(idx_hbm.at[pl.ds(base, CH)], idxp_v.at[buf])
            for r in range(CH):
                idx_v[buf, pl.ds(r * K, K)] = idxp_v[buf, r, pl.ds(0, K)]
            pltpu.async_copy(h_hbm.at[idx_v.at[buf, pl.ds(0, half)]],
                             rows_v.at[buf, pl.ds(0, half)], gsem[buf])
            pltpu.async_copy(h_hbm.at[idx_v.at[buf, pl.ds(half, half)]],
                             rows_v.at[buf, pl.ds(half, half)], gsem[buf])
            pltpu.async_copy(h_hbm.at[pl.ds(base, CH)], hrow_v.at[buf],
                             hsem[buf])

        def wait_fetch(t, buf):
            base = w0 + t * CH
            pltpu.make_async_copy(h_hbm.at[idx_v.at[buf, pl.ds(0, half)]],
                                  rows_v.at[buf, pl.ds(0, half)],
                                  gsem[buf]).wait()
            pltpu.make_async_copy(h_hbm.at[idx_v.at[buf, pl.ds(half, half)]],
                                  rows_v.at[buf, pl.ds(half, half)],
                                  gsem[buf]).wait()
            pltpu.make_async_copy(h_hbm.at[pl.ds(base, CH)],
                                  hrow_v.at[buf], hsem[buf]).wait()

        def compute(t, buf):
            base = w0 + t * CH
            wait_fetch(t, buf)
            for r in range(CH):
                for c in range(C // LANES):
                    sl = pl.ds(c * LANES, LANES)
                    v = [rows_v[buf, r * K + k, sl] for k in range(K)]
                    while len(v) > 1:
                        nxt = [jnp.maximum(v[i], v[i + 1])
                               for i in range(0, len(v) - 1, 2)]
                        if len(v) % 2:
                            nxt.append(v[-1])
                        v = nxt
                    out_v[r, sl] = v[0] - hrow_v[buf, r, sl]
            pltpu.sync_copy(out_v, rel_hbm.at[pl.ds(base, CH)])

        fetch(0, 0)

        def pair(g, carry):
            t0 = g * 2
            fetch(t0 + 1, 1)
            compute(t0, 0)
            fetch(jnp.minimum(t0 + 2, nch - 1), 0)
            compute(t0 + 1, 1)
            return carry

        lax.fori_loop(0, nch // 2, pair, None)
        wait_fetch(nch - 1, 0)

    return gather_rel


def _gather_rel(h_flat, idx_flat):
    return _make_gather_rel()(h_flat, idx_flat)


def _gconv_head_body(h_ref, rel_ref, res_ref, gwh_ref, gwr_ref, gb_ref,
                     fc2_w_ref, fc2_b_ref, hw_ref, hb_ref, out_ref):
    h = h_ref[...]
    rel = rel_ref[...]
    pre = (jnp.dot(h, gwh_ref[...], preferred_element_type=jnp.float32)
           + jnp.dot(rel, gwr_ref[...], preferred_element_type=jnp.float32)
           + gb_ref[...])
    g = jax.nn.gelu(pre)
    h2 = jnp.dot(g, fc2_w_ref[...],
                 preferred_element_type=jnp.float32) + fc2_b_ref[...]
    x2 = h2 + res_ref[...]
    out_ref[...] = jnp.dot(x2, hw_ref[...],
                           preferred_element_type=jnp.float32) + hb_ref[...]


def _gconv_head(h, rel, res, gwh, gwr, gb2, fc2_w, fc2_b2, hw_pad, hb_pad):
    M = B * N
    return pl.pallas_call(
        _gconv_head_body,
        grid=(M // ROWS_D,),
        in_specs=[
            pl.BlockSpec((ROWS_D, C), lambda i: (i, 0)),
            pl.BlockSpec((ROWS_D, C), lambda i: (i, 0)),
            pl.BlockSpec((ROWS_D, C), lambda i: (i, 0)),
            pl.BlockSpec((C, C), lambda i: (0, 0)),
            pl.BlockSpec((C, C), lambda i: (0, 0)),
            pl.BlockSpec((1, C), lambda i: (0, 0)),
            pl.BlockSpec((C, C), lambda i: (0, 0)),
            pl.BlockSpec((1, C), lambda i: (0, 0)),
            pl.BlockSpec((C, CLS_PAD), lambda i: (0, 0)),
            pl.BlockSpec((1, CLS_PAD), lambda i: (0, 0)),
        ],
        out_specs=pl.BlockSpec((ROWS_D, CLS_PAD), lambda i: (i, 0)),
        out_shape=jax.ShapeDtypeStruct((M, CLS_PAD), jnp.float32),
    )(h, rel, res, gwh, gwr, gb2, fc2_w, fc2_b2, hw_pad, hb_pad)


CLS_BLK = 128


def _lsm_body(l_ref, out_ref):
    l = l_ref[0]
    m = jnp.max(l, axis=0, keepdims=True)
    s = jnp.sum(jnp.exp(l - m), axis=0, keepdims=True)
    out_ref[0] = l - m - jnp.log(s)


def _log_softmax_n(logits):
    return pl.pallas_call(
        _lsm_body,
        grid=(B, CLS_PAD // CLS_BLK),
        in_specs=[pl.BlockSpec((1, N, CLS_BLK), lambda b, j: (b, 0, j))],
        out_specs=pl.BlockSpec((1, N, CLS_BLK), lambda b, j: (b, 0, j)),
        out_shape=jax.ShapeDtypeStruct((B, N, CLS_PAD), jnp.float32),
    )(logits)


@jax.jit
def kernel(x, cpe_w, cpe_b, fc1_w, fc1_b, gconv_w, gconv_b, fc2_w, fc2_b,
           head_w, head_b):
    cpe_b2 = cpe_b.reshape(1, C)
    fc1_b2 = fc1_b.reshape(1, C)
    gb2 = gconv_b.reshape(1, C)
    fc2_b2 = fc2_b.reshape(1, C)
    hw_pad = jnp.zeros((C, CLS_PAD), jnp.float32).at[:, :NUM_CLASSES].set(head_w)
    hb_pad = jnp.zeros((1, CLS_PAD), jnp.float32).at[:, :NUM_CLASSES].set(
        head_b.reshape(1, NUM_CLASSES))

    xc, h = _cpe_fc1(x, cpe_w, cpe_b2, fc1_w, fc1_b2)
    gwh = gconv_w[:C, :]
    gwr = gconv_w[C:, :]

    idx0 = _knn_topk(h[0])
    rel0 = _gather_rel(h[0], idx0)
    idx1 = _knn_topk(h[1])
    rel1 = _gather_rel(h[1], idx1)
    rel_flat = jnp.concatenate([rel0, rel1], axis=0)
    logits = _gconv_head(h.reshape(B * N, C), rel_flat,
                         xc.reshape(B * N, C), gwh, gwr, gb2,
                         fc2_w, fc2_b2, hw_pad, hb_pad)
    out = _log_softmax_n(logits.reshape(B, N, CLS_PAD))
    return out[:, :, :NUM_CLASSES]


# speedup vs baseline: 1.0735x; 1.0735x over previous
import functools

import jax
import jax.numpy as jnp
from jax import lax
from jax.experimental import pallas as pl
from jax.experimental.pallas import tpu as pltpu
from jax.experimental.pallas import tpu_sc as plsc

B, N, C, K, NUM_CLASSES = 2, 4096, 128, 16, 1000
CLS_PAD = 1024
ROWS_B = 256
ROWS_D = 512
BIG = 3.0e38


def _cpe_fc1_body(x_ref, cpe_w_ref, cpe_b_ref, fc1_w_ref, fc1_b_ref,
                  xc_ref, h0_ref, h1_ref):
    b = pl.program_id(0)
    x = x_ref[0]
    w0 = cpe_w_ref[0:1, :]
    w1 = cpe_w_ref[1:2, :]
    w2 = cpe_w_ref[2:3, :]
    zrow = jnp.zeros((1, C), jnp.float32)
    xm1 = jnp.concatenate([zrow, x[:-1, :]], axis=0)
    xp1 = jnp.concatenate([x[1:, :], zrow], axis=0)
    conv = xm1 * w0 + x * w1 + xp1 * w2 + cpe_b_ref[...]
    xc = x + conv
    xc_ref[0] = xc
    h = jnp.dot(xc, fc1_w_ref[...],
                preferred_element_type=jnp.float32) + fc1_b_ref[...]

    @pl.when(b == 0)
    def _():
        h0_ref[...] = h

    @pl.when(b == 1)
    def _():
        h1_ref[...] = h


def _cpe_fc1(x, cpe_w, cpe_b2, fc1_w, fc1_b2):
    return pl.pallas_call(
        _cpe_fc1_body,
        grid=(B,),
        in_specs=[
            pl.BlockSpec((1, N, C), lambda b: (b, 0, 0)),
            pl.BlockSpec((3, C), lambda b: (0, 0)),
            pl.BlockSpec((1, C), lambda b: (0, 0)),
            pl.BlockSpec((C, C), lambda b: (0, 0)),
            pl.BlockSpec((1, C), lambda b: (0, 0)),
        ],
        out_specs=[
            pl.BlockSpec((1, N, C), lambda b: (b, 0, 0)),
            pl.BlockSpec((N, C), lambda b: (0, 0)),
            pl.BlockSpec((N, C), lambda b: (0, 0)),
        ],
        out_shape=[
            jax.ShapeDtypeStruct((B, N, C), jnp.float32),
            jax.ShapeDtypeStruct((N, C), jnp.float32),
            jax.ShapeDtypeStruct((N, C), jnp.float32),
        ],
    )(x, cpe_w, cpe_b2, fc1_w, fc1_b2)


def _topk_body(hrow_ref, hfull_ref, idx_ref, d_ref):
    hb = hrow_ref[...]
    h = hfull_ref[...]
    sqcol = jnp.sum(h * h, axis=1, keepdims=True)
    lhs = jnp.concatenate([hb * (-2.0), jnp.ones((ROWS_B, 1), jnp.float32)],
                          axis=1)
    rhs = jnp.concatenate([h, sqcol], axis=1)
    e = lax.dot_general(lhs, rhs, (((1,), (1,)), ((), ())),
                        preferred_element_type=jnp.float32)
    d_ref[...] = e
    iota = lax.broadcasted_iota(jnp.int32, (ROWS_B, N), 1).astype(jnp.float32)
    cols = []
    for _ in range(K // 2):
        d = d_ref[...]
        m1 = jnp.min(d, axis=1, keepdims=True)
        c1 = jnp.min(jnp.where(d <= m1, iota, jnp.float32(1e9)),
                     axis=1, keepdims=True)
        d = jnp.where(iota == c1, BIG, d)
        m2 = jnp.min(d, axis=1, keepdims=True)
        c2 = jnp.min(jnp.where(d <= m2, iota, jnp.float32(1e9)),
                     axis=1, keepdims=True)
        d_ref[...] = jnp.where(iota == c2, BIG, d)
        cols += [c1, c2]
    pad = jnp.zeros((ROWS_B, 128 - K), jnp.float32)
    idx_mat = jnp.concatenate(cols + [pad], axis=1).astype(jnp.int32)
    idx_ref[...] = idx_mat


NH = N // 2


def _knn_topk_half(h, half):
    hb = NH // ROWS_B
    return pl.pallas_call(
        _topk_body,
        grid=(hb,),
        in_specs=[
            pl.BlockSpec((ROWS_B, C), lambda i: (i + half * hb, 0)),
            pl.BlockSpec((N, C), lambda i: (0, 0)),
        ],
        out_specs=pl.BlockSpec((ROWS_B, 128), lambda i: (i, 0)),
        out_shape=jax.ShapeDtypeStruct((NH, 128), jnp.int32),
        scratch_shapes=[pltpu.VMEM((ROWS_B, N), jnp.float32)],
    )(h, h)


CH = 8
LANES = 16


@functools.lru_cache(maxsize=None)
def _make_gather_rel(nrows, h_off):
    info = plsc.get_sparse_core_info()
    num_cores = info.num_cores
    nw = num_cores * info.num_subcores
    rows_w = nrows // nw

    nch = rows_w // CH

    @functools.partial(
        pl.kernel,
        out_type=jax.ShapeDtypeStruct((nrows, C), jnp.float32),
        mesh=plsc.VectorSubcoreMesh(core_axis_name="c",
                                    subcore_axis_name="s"),
        scratch_types=[
            pltpu.VMEM((2, CH, 128), jnp.int32),
            pltpu.VMEM((2, CH * K), jnp.int32),
            pltpu.VMEM((2, CH * K, C), jnp.float32),
            pltpu.VMEM((2, CH, C), jnp.float32),
            pltpu.VMEM((CH, C), jnp.float32),
            pltpu.SemaphoreType.DMA,
            pltpu.SemaphoreType.DMA,
            pltpu.SemaphoreType.DMA,
            pltpu.SemaphoreType.DMA,
        ],
    )
    def gather_rel(h_hbm, idx_hbm, rel_hbm, idxp_v, idx_v, rows_v, hrow_v,
                   out_v, gsem0, gsem1, hsem0, hsem1):
        gsem = (gsem0, gsem1)
        hsem = (hsem0, hsem1)
        wid = lax.axis_index("s") * num_cores + lax.axis_index("c")
        w0 = wid * rows_w

        def fetch(t, buf):
            base = w0 + t * CH
            pltpu.sync_copy(idx_hbm.at[pl.ds(base, CH)], idxp_v.at[buf])
            for r in range(CH):
                idx_v[buf, pl.ds(r * K, K)] = idxp_v[buf, r, pl.ds(0, K)]
            pltpu.async_copy(h_hbm.at[idx_v.at[buf]], rows_v.at[buf],
                             gsem[buf])
            pltpu.async_copy(h_hbm.at[pl.ds(h_off + base, CH)],
                             hrow_v.at[buf], hsem[buf])

        def compute(t, buf):
            base = w0 + t * CH
            pltpu.make_async_copy(h_hbm.at[idx_v.at[buf]], rows_v.at[buf],
                                  gsem[buf]).wait()
            pltpu.make_async_copy(h_hbm.at[pl.ds(h_off + base, CH)],
                                  hrow_v.at[buf], hsem[buf]).wait()
            for r in range(CH):
                for c in range(C // LANES):
                    sl = pl.ds(c * LANES, LANES)
                    acc = rows_v[buf, r * K, sl]
                    for k in range(1, K):
                        acc = jnp.maximum(acc, rows_v[buf, r * K + k, sl])
                    out_v[r, sl] = acc - hrow_v[buf, r, sl]
            pltpu.sync_copy(out_v, rel_hbm.at[pl.ds(base, CH)])

        fetch(0, 0)

        def pair(g, carry):
            t0 = g * 2
            fetch(t0 + 1, 1)
            compute(t0, 0)
            fetch(jnp.minimum(t0 + 2, nch - 1), 0)
            compute(t0 + 1, 1)
            return carry

        lax.fori_loop(0, nch // 2, pair, None)
        last = w0 + (nch - 1) * CH
        pltpu.make_async_copy(h_hbm.at[idx_v.at[0]], rows_v.at[0],
                              gsem[0]).wait()
        pltpu.make_async_copy(h_hbm.at[pl.ds(h_off + last, CH)],
                              hrow_v.at[0], hsem[0]).wait()

    return gather_rel


def _gather_rel(h_full, idx_half, half):
    return _make_gather_rel(NH, half * NH)(h_full, idx_half)


def _gconv_head_body(h_ref, rel_ref, res_ref, gwh_ref, gwr_ref, gb_ref,
                     fc2_w_ref, fc2_b_ref, hw_ref, hb_ref, out_ref):
    h = h_ref[...]
    rel = rel_ref[...]
    pre = (jnp.dot(h, gwh_ref[...], preferred_element_type=jnp.float32)
           + jnp.dot(rel, gwr_ref[...], preferred_element_type=jnp.float32)
           + gb_ref[...])
    g = jax.nn.gelu(pre)
    h2 = jnp.dot(g, fc2_w_ref[...],
                 preferred_element_type=jnp.float32) + fc2_b_ref[...]
    x2 = h2 + res_ref[...]
    out_ref[...] = jnp.dot(x2, hw_ref[...],
                           preferred_element_type=jnp.float32) + hb_ref[...]


def _gconv_head(h, rel, res, gwh, gwr, gb2, fc2_w, fc2_b2, hw_pad, hb_pad):
    M = B * N
    return pl.pallas_call(
        _gconv_head_body,
        grid=(M // ROWS_D,),
        in_specs=[
            pl.BlockSpec((ROWS_D, C), lambda i: (i, 0)),
            pl.BlockSpec((ROWS_D, C), lambda i: (i, 0)),
            pl.BlockSpec((ROWS_D, C), lambda i: (i, 0)),
            pl.BlockSpec((C, C), lambda i: (0, 0)),
            pl.BlockSpec((C, C), lambda i: (0, 0)),
            pl.BlockSpec((1, C), lambda i: (0, 0)),
            pl.BlockSpec((C, C), lambda i: (0, 0)),
            pl.BlockSpec((1, C), lambda i: (0, 0)),
            pl.BlockSpec((C, CLS_PAD), lambda i: (0, 0)),
            pl.BlockSpec((1, CLS_PAD), lambda i: (0, 0)),
        ],
        out_specs=pl.BlockSpec((ROWS_D, CLS_PAD), lambda i: (i, 0)),
        out_shape=jax.ShapeDtypeStruct((M, CLS_PAD), jnp.float32),
    )(h, rel, res, gwh, gwr, gb2, fc2_w, fc2_b2, hw_pad, hb_pad)


CLS_BLK = 128


def _lsm_body(l_ref, out_ref):
    l = l_ref[0]
    m = jnp.max(l, axis=0, keepdims=True)
    s = jnp.sum(jnp.exp(l - m), axis=0, keepdims=True)
    out_ref[0] = l - m - jnp.log(s)


def _log_softmax_n(logits):
    return pl.pallas_call(
        _lsm_body,
        grid=(B, CLS_PAD // CLS_BLK),
        in_specs=[pl.BlockSpec((1, N, CLS_BLK), lambda b, j: (b, 0, j))],
        out_specs=pl.BlockSpec((1, N, CLS_BLK), lambda b, j: (b, 0, j)),
        out_shape=jax.ShapeDtypeStruct((B, N, CLS_PAD), jnp.float32),
    )(logits)


@jax.jit
def kernel(x, cpe_w, cpe_b, fc1_w, fc1_b, gconv_w, gconv_b, fc2_w, fc2_b,
           head_w, head_b):
    cpe_b2 = cpe_b.reshape(1, C)
    fc1_b2 = fc1_b.reshape(1, C)
    gb2 = gconv_b.reshape(1, C)
    fc2_b2 = fc2_b.reshape(1, C)
    hw_pad = jnp.zeros((C, CLS_PAD), jnp.float32).at[:, :NUM_CLASSES].set(head_w)
    hb_pad = jnp.zeros((1, CLS_PAD), jnp.float32).at[:, :NUM_CLASSES].set(
        head_b.reshape(1, NUM_CLASSES))

    xc, h0, h1 = _cpe_fc1(x, cpe_w, cpe_b2, fc1_w, fc1_b2)
    gwh = gconv_w[:C, :]
    gwr = gconv_w[C:, :]

    i0a = _knn_topk_half(h0, 0)
    r0a = _gather_rel(h0, i0a, 0)
    i0b = _knn_topk_half(h0, 1)
    r0b = _gather_rel(h0, i0b, 1)
    i1a = _knn_topk_half(h1, 0)
    r1a = _gather_rel(h1, i1a, 0)
    i1b = _knn_topk_half(h1, 1)
    r1b = _gather_rel(h1, i1b, 1)
    rel_flat = jnp.concatenate([r0a, r0b, r1a, r1b], axis=0)
    h_flat = jnp.concatenate([h0, h1], axis=0)
    logits = _gconv_head(h_flat, rel_flat,
                         xc.reshape(B * N, C), gwh, gwr, gb2,
                         fc2_w, fc2_b2, hw_pad, hb_pad)
    out = _log_softmax_n(logits.reshape(B, N, CLS_PAD))
    return out[:, :, :NUM_CLASSES]
